# baseline (speedup 1.0000x reference)
import functools

import jax
import jax.numpy as jnp
from jax import lax
from jax.experimental import pallas as pl
from jax.experimental.pallas import tpu as pltpu
from jax.experimental.pallas import tpu_sc as plsc

N_NODES = 100000
E_EDGES = 1600000
NC, NS, L = 2, 16, 16
NP = 100352
NPT = NP // NS
E_PER_TILE = E_EDGES // NS
E_PER_WORKER = E_EDGES // (NC * NS)
CH = 10000

_MESH = plsc.VectorSubcoreMesh(
    core_axis_name="c", subcore_axis_name="s", num_cores=NC, num_subcores=NS)


@functools.partial(
    pl.kernel,
    out_type=(
        jax.ShapeDtypeStruct((NC, NS, NP), jnp.float32),
        jax.ShapeDtypeStruct((NC, NS, NP), jnp.float32),
    ),
    mesh=_MESH,
    compiler_params=pltpu.CompilerParams(needs_layout_passes=False),
    scratch_types=[
        pltpu.VMEM((NP,), jnp.float32),
        pltpu.VMEM((CH,), jnp.int32),
        pltpu.VMEM((CH,), jnp.float32),
        pltpu.VMEM_SHARED((NP,), jnp.float32),
    ],
)
def _sc_kernel(src_hbm, dst_hbm, degp_hbm, wp_hbm,
               part_v, idx_v, val_v, dis_sh):
    c = lax.axis_index("c")
    s = lax.axis_index("s")
    ones16 = jnp.full((L,), 1.0, jnp.float32)
    zeros16 = jnp.zeros((L,), jnp.float32)

    def zero_part(_unused):
        def body(i, carry):
            part_v[pl.ds(i * L, L)] = zeros16
            return carry
        lax.fori_loop(0, NP // L, body, 0)

    zero_part(None)

    def hchunk(i, carry):
        pltpu.sync_copy(dst_hbm.at[pl.ds(s * E_PER_TILE + i * CH, CH)], idx_v)

        def hbody(j, carry2):
            ix = idx_v[pl.ds(j * L, L)]
            plsc.addupdate_scatter(part_v, [ix], ones16)
            return carry2

        lax.fori_loop(0, CH // L, hbody, 0)
        return carry

    lax.fori_loop(0, E_PER_TILE // CH, hchunk, 0)
    pltpu.sync_copy(part_v, degp_hbm.at[c, s])
    plsc.subcore_barrier()

    for t in range(NS):
        pltpu.sync_copy(degp_hbm.at[c, t, pl.ds(s * NPT, NPT)],
                        part_v.at[pl.ds(t * NPT, NPT)])

    def dbody(j, carry):
        acc = part_v[pl.ds(j * L, L)]
        for t in range(1, NS):
            acc = acc + part_v[pl.ds(t * NPT + j * L, L)]
        d = acc + 1.0
        h = 0.5 * d
        bits = lax.bitcast_convert_type(d, jnp.int32)
        y = lax.bitcast_convert_type(
            0x5F3759DF - lax.shift_right_logical(bits, 1), jnp.float32)
        y = y * (1.5 - h * y * y)
        y = y * (1.5 - h * y * y)
        y = y * (1.5 - h * y * y)
        part_v[pl.ds(j * L, L)] = y
        return carry

    lax.fori_loop(0, NPT // L, dbody, 0)
    pltpu.sync_copy(part_v.at[pl.ds(0, NPT)], dis_sh.at[pl.ds(s * NPT, NPT)])
    plsc.subcore_barrier()

    zero_part(None)
    wid = s * NC + c

    def echunk(i, carry):
        base = wid * E_PER_WORKER + i * CH
        pltpu.sync_copy(dst_hbm.at[pl.ds(base, CH)], idx_v)
        pltpu.sync_copy(dis_sh.at[idx_v], val_v)
        pltpu.sync_copy(src_hbm.at[pl.ds(base, CH)], idx_v)

        def ebody(j, carry2):
            ix = idx_v[pl.ds(j * L, L)]
            v = val_v[pl.ds(j * L, L)]
            plsc.addupdate_scatter(part_v, [ix], v)
            return carry2

        lax.fori_loop(0, CH // L, ebody, 0)
        return carry

    lax.fori_loop(0, E_PER_WORKER // CH, echunk, 0)
    pltpu.sync_copy(part_v, wp_hbm.at[c, s])


def _b16(a):
    return a.astype(jnp.bfloat16).astype(jnp.float32)


def _tc_tail_body(degp_ref, wp_ref, x3_ref, W0_ref, Wi_ref, Wh_ref,
                  b_ref, W1_ref, b1_ref, W2_ref, b2_ref, out_ref):
    dotH = functools.partial(jnp.dot, precision=lax.Precision.HIGHEST)

    W = W0_ref[...]
    h = W
    cst = jnp.zeros_like(W)
    Wi = _b16(Wi_ref[...])
    Wh = _b16(Wh_ref[...])
    b = b_ref[...]
    for _ in range(8):
        gates = dotH(_b16(W), Wi) + dotH(_b16(h), Wh) + b
        i_g, f_g, g_g, o_g = jnp.split(gates, 4, axis=-1)
        cst = jax.nn.sigmoid(f_g) * cst + jax.nn.sigmoid(i_g) * jnp.tanh(g_g)
        h = jax.nn.sigmoid(o_g) * jnp.tanh(cst)
        W = h

    deg = jnp.sum(degp_ref[...], axis=0) + 1.0
    dis = lax.rsqrt(deg)
    w = jnp.sum(wp_ref[...], axis=0)
    coef = dis * w + dis * dis

    xb = _b16(x3_ref[...])
    s0 = jnp.sum(coef * xb[0])
    s1 = jnp.sum(coef * xb[1])
    s2 = jnp.sum(coef * xb[2])

    Wb = _b16(W)
    g = (s0 * Wb[0] + s1 * Wb[1] + s2 * Wb[2]) * (1.0 / N_NODES)
    hid = jnp.maximum(
        dotH(_b16(g[None, :]), _b16(W1_ref[...])) + b1_ref[...][None, :], 0.0)
    out_ref[...] = (dotH(_b16(hid), _b16(W2_ref[...]))
                    + b2_ref[...][None, :])


_tc_tail = pl.pallas_call(
    _tc_tail_body,
    out_shape=jax.ShapeDtypeStruct((1, 2), jnp.float32),
)


def kernel(x_seq, edge_index_seq, W0, lstm_Wi, lstm_Wh, lstm_b, W1, b1, W2, b2):
    src = edge_index_seq[-1, 0].astype(jnp.int32)
    dst = edge_index_seq[-1, 1].astype(jnp.int32)
    x3 = jnp.pad(x_seq[-1].T, ((0, 0), (0, NP - N_NODES)))
    x3 = x3.reshape(3, NP // 128, 128)

    degp, wp = _sc_kernel(src, dst)

    r = (NP // 128, 128)
    return _tc_tail(degp[0].reshape((NS,) + r), wp.reshape((NC * NS,) + r),
                    x3, W0, lstm_Wi, lstm_Wh, lstm_b, W1, b1, W2, b2)
